# Initial kernel scaffold; baseline (speedup 1.0000x reference)
import jax
import jax.numpy as jnp
from jax.experimental import pallas as pl


def kernel(x, edge_index, edge_type, node_graph_ids, W_msg, b_msg, W_ih, W_hh, b_ih, b_hh, W_i, W_j, b_j):
    raise NotImplementedError("write your pallas kernel here")


# speedup vs baseline: 24.6633x; 24.6633x over previous
import functools

import jax
import jax.numpy as jnp
from jax import lax
from jax.experimental import pallas as pl
from jax.experimental.pallas import tpu as pltpu
from jax.experimental.pallas import tpu_sc as plsc

N = 10000
E = 320000
D = 128
M = 128
T = 4
NG = 64

NC = 2
NS = 16
NW = NC * NS
EPW = E // NW
K = 80
G = 2000
NGRP = EPW // G
CPG = G // K
ZR = 40
NZC = N // ZR
WR = 2000

BN = 1000
NB = N // BN


def _msgs_body(x_ref, w_ref, b_ref, o_ref):
    t = pl.program_id(0)
    w = w_ref[0]
    b = b_ref[pl.ds(t, 1), :]
    acc = lax.dot_general(x_ref[...], w, (((1,), (1,)), ((), ())),
                          preferred_element_type=jnp.float32)
    o_ref[0] = acc + b


def _build_msg_table(x, W_msg, b_msg):
    return pl.pallas_call(
        _msgs_body,
        grid=(T, NB),
        in_specs=[
            pl.BlockSpec((BN, D), lambda t, i: (i, 0)),
            pl.BlockSpec((1, M, D), lambda t, i: (t, 0, 0)),
            pl.BlockSpec((T, M), lambda t, i: (0, 0)),
        ],
        out_specs=pl.BlockSpec((1, BN, M), lambda t, i: (t, i, 0)),
        out_shape=jax.ShapeDtypeStruct((T, N, M), jnp.float32),
    )(x, W_msg, b_msg)


def _sc_body(src_hbm, et_hbm, dst_hbm, table_hbm, out_hbm,
             src_v, et_v, idx_v, dst1, dst2, rows, zbuf, acc, sem):
    c = lax.axis_index("c")
    s = lax.axis_index("s")
    wid = s * NC + c
    base = wid * EPW

    def _zfill(i, carry):
        r = i // 8
        col = (i % 8) * 16
        zbuf[r, pl.ds(col, 16)] = jnp.zeros((16,), jnp.float32)
        return carry
    lax.fori_loop(0, ZR * 8, _zfill, 0)

    def _zacc(k, carry):
        cid = k * NS + s
        @pl.when(cid < NZC)
        def _():
            pltpu.sync_copy(zbuf, acc.at[pl.ds(cid * ZR, ZR)])
        return carry
    lax.fori_loop(0, (NZC + NS - 1) // NS, _zacc, 0)
    plsc.subcore_barrier()

    def _group(g, carry):
        gbase = base + g * G
        pltpu.sync_copy(src_hbm.at[pl.ds(gbase, G)], src_v)
        pltpu.sync_copy(et_hbm.at[pl.ds(gbase, G)], et_v)
        pltpu.sync_copy(dst_hbm.at[pl.ds(gbase, G)], dst1)

        def _mkidx(i, c2):
            off = i * 16
            sl = pl.ds(off, 16)
            idx_v[sl] = et_v[sl] * N + src_v[sl]
            dst2[off // K, pl.ds(off % K, 16)] = dst1[sl]
            return c2
        lax.fori_loop(0, G // 16, _mkidx, 0)

        def _chunk(j, c2):
            pltpu.sync_copy(table_hbm.at[idx_v.at[pl.ds(j * K, K)]], rows)
            pltpu.sync_copy(rows, acc.at[dst2.at[j]], add=True)
            return c2
        lax.fori_loop(0, CPG, _chunk, 0)
        return carry
    lax.fori_loop(0, NGRP, _group, 0)
    plsc.subcore_barrier()

    @pl.when(s < N // WR)
    def _():
        sl = pl.ds(s * WR, WR)
        pltpu.sync_copy(acc.at[sl], out_hbm.at[c].at[sl])


def _sc_scatter(src, et, dst, table_flat):
    mesh = plsc.VectorSubcoreMesh(core_axis_name="c", subcore_axis_name="s")
    f = functools.partial(
        pl.kernel,
        out_type=jax.ShapeDtypeStruct((NC, N, M), jnp.float32),
        mesh=mesh,
        scratch_types=[
            pltpu.VMEM((G,), jnp.int32),
            pltpu.VMEM((G,), jnp.int32),
            pltpu.VMEM((G,), jnp.int32),
            pltpu.VMEM((G,), jnp.int32),
            pltpu.VMEM((CPG, K), jnp.int32),
            pltpu.VMEM((K, M), jnp.float32),
            pltpu.VMEM((ZR, M), jnp.float32),
            pltpu.VMEM_SHARED((N, M), jnp.float32),
            pltpu.SemaphoreType.DMA,
        ],
    )(_sc_body)
    return f(src, et, dst, table_flat)


def _update_body(part_ref, x_ref, gid_ref, wih_ref, whh_ref, bih_ref,
                 bhh_ref, wi_ref, wj_ref, bj_ref, rd_ref, phis_ref):
    i = pl.program_id(0)
    m = part_ref[0] + part_ref[1]
    h = x_ref[...]

    gi = lax.dot_general(m, wih_ref[...], (((1,), (1,)), ((), ())),
                         preferred_element_type=jnp.float32) + bih_ref[0][None, :]
    gh = lax.dot_general(h, whh_ref[...], (((1,), (1,)), ((), ())),
                         preferred_element_type=jnp.float32) + bhh_ref[0][None, :]
    r = jax.nn.sigmoid(gi[:, :D] + gh[:, :D])
    z = jax.nn.sigmoid(gi[:, D:2 * D] + gh[:, D:2 * D])
    n = jnp.tanh(gi[:, 2 * D:] + r * gh[:, 2 * D:])
    hT = (1.0 - z) * n + z * h

    wi = wi_ref[...]
    wj = wj_ref[...]
    a = (lax.dot_general(hT, wi[:, :D], (((1,), (1,)), ((), ())),
                         preferred_element_type=jnp.float32)
         + lax.dot_general(h, wi[:, D:], (((1,), (1,)), ((), ())),
                           preferred_element_type=jnp.float32))
    b = (lax.dot_general(hT, wj[:, :D], (((1,), (1,)), ((), ())),
                         preferred_element_type=jnp.float32)
         + lax.dot_general(h, wj[:, D:], (((1,), (1,)), ((), ())),
                           preferred_element_type=jnp.float32)
         + bj_ref[0][None, :])
    rd = jax.nn.sigmoid(a) * jnp.tanh(b)
    rd_ref[...] = rd

    ids = gid_ref[...].reshape(1, BN)
    iota = lax.broadcasted_iota(jnp.int32, (NG, BN), 0)
    onehot = jnp.where(ids == iota, 1.0, 0.0)
    pb = lax.dot_general(onehot, rd, (((1,), (0,)), ((), ())),
                         preferred_element_type=jnp.float32)

    @pl.when(i == 0)
    def _init():
        phis_ref[...] = pb

    @pl.when(i > 0)
    def _acc():
        phis_ref[...] += pb


def _update(partials, x, gids3, W_ih, W_hh, b_ih, b_hh, W_i, W_j, b_j):
    full = lambda shape: pl.BlockSpec(shape, lambda i: tuple(0 for _ in shape))
    return pl.pallas_call(
        _update_body,
        grid=(NB,),
        in_specs=[
            pl.BlockSpec((NC, BN, M), lambda i: (0, i, 0)),
            pl.BlockSpec((BN, D), lambda i: (i, 0)),
            pl.BlockSpec((1, 1, BN), lambda i: (i, 0, 0)),
            full((3 * D, M)),
            full((3 * D, D)),
            full((1, 3 * D)),
            full((1, 3 * D)),
            full((D, 2 * D)),
            full((D, 2 * D)),
            full((1, D)),
        ],
        out_specs=[
            pl.BlockSpec((BN, D), lambda i: (i, 0)),
            pl.BlockSpec((NG, D), lambda i: (0, 0)),
        ],
        out_shape=[
            jax.ShapeDtypeStruct((N, D), jnp.float32),
            jax.ShapeDtypeStruct((NG, D), jnp.float32),
        ],
    )(partials, x, gids3, W_ih, W_hh, b_ih, b_hh, W_i, W_j, b_j)


def kernel(x, edge_index, edge_type, node_graph_ids, W_msg, b_msg,
           W_ih, W_hh, b_ih, b_hh, W_i, W_j, b_j):
    table = _build_msg_table(x, W_msg, b_msg)
    table_flat = table.reshape(T * N, M)

    partials = _sc_scatter(edge_index[0], edge_type, edge_index[1],
                           table_flat)

    gids3 = node_graph_ids.reshape(NB, 1, BN)
    rd, phis = _update(partials, x, gids3, W_ih, W_hh,
                       b_ih.reshape(1, -1), b_hh.reshape(1, -1),
                       W_i, W_j, b_j.reshape(1, -1))
    return (rd, phis)


# speedup vs baseline: 30.8458x; 1.2507x over previous
import functools

import jax
import jax.numpy as jnp
from jax import lax
from jax.experimental import pallas as pl
from jax.experimental.pallas import tpu as pltpu
from jax.experimental.pallas import tpu_sc as plsc

N = 10000
E = 320000
D = 128
M = 128
T = 4
NG = 64

NC = 2
NS = 16
NW = NC * NS
EPW = E // NW
K = 80
NCH = EPW // K
NPAIR = (NCH - 1) // 2
ZR = 40
NZC = N // ZR
WR = 2000
EB = E // 128

BN = 1000
NB = N // BN


def _msgs_body(x_ref, w_ref, b_ref, o_ref):
    t = pl.program_id(0)
    w = w_ref[0]
    b = b_ref[pl.ds(t, 1), :]
    acc = lax.dot_general(x_ref[...], w, (((1,), (1,)), ((), ())),
                          preferred_element_type=jnp.float32)
    o_ref[0] = acc + b


def _build_msg_table(x, W_msg, b_msg):
    return pl.pallas_call(
        _msgs_body,
        grid=(T, NB),
        in_specs=[
            pl.BlockSpec((BN, D), lambda t, i: (i, 0)),
            pl.BlockSpec((1, M, D), lambda t, i: (t, 0, 0)),
            pl.BlockSpec((T, M), lambda t, i: (0, 0)),
        ],
        out_specs=pl.BlockSpec((1, BN, M), lambda t, i: (t, i, 0)),
        out_shape=jax.ShapeDtypeStruct((T, N, M), jnp.float32),
    )(x, W_msg, b_msg)


def _idx_body(src_ref, et_ref, o_ref):
    o_ref[...] = et_ref[...] * N + src_ref[...]


def _build_idx(src, et):
    out = pl.pallas_call(
        _idx_body,
        out_shape=jax.ShapeDtypeStruct((EB, 128), jnp.int32),
    )(src.reshape(EB, 128), et.reshape(EB, 128))
    return out.reshape(E)


def _sc_body(idx_hbm, dst_hbm, table_hbm, out_hbm,
             idx_v, dst_v, rows0, rows1, zbuf, acc,
             sg0, sg1, ss0, ss1):
    c = lax.axis_index("c")
    s = lax.axis_index("s")
    wid = s * NC + c
    base = wid * EPW

    pltpu.sync_copy(idx_hbm.at[pl.ds(base, EPW)], idx_v)
    pltpu.sync_copy(dst_hbm.at[pl.ds(base, EPW)], dst_v)

    def _zfill(i, carry):
        zbuf[i // 8, pl.ds((i % 8) * 16, 16)] = jnp.zeros((16,), jnp.float32)
        return carry
    lax.fori_loop(0, ZR * 8, _zfill, 0)

    def _zacc(k, carry):
        cid = k * NS + s
        @pl.when(cid < NZC)
        def _():
            pltpu.sync_copy(zbuf, acc.at[pl.ds(cid * ZR, ZR)])
        return carry
    lax.fori_loop(0, (NZC + NS - 1) // NS, _zacc, 0)
    plsc.subcore_barrier()

    def _gsrc(j):
        return table_hbm.at[idx_v.at[pl.ds(j * K, K)]]

    def _sdst(j):
        return acc.at[dst_v.at[pl.ds(j * K, K)]]

    pltpu.async_copy(_gsrc(0), rows0, sg0)

    def _pair(jj, carry):
        j0 = 2 * jj
        j1 = j0 + 1
        pltpu.make_async_copy(_gsrc(j0), rows0, sg0).wait()
        @pl.when(jj > 0)
        def _():
            pltpu.make_async_copy(rows1, _sdst(j1 - 2), ss1).wait()
        pltpu.async_copy(_gsrc(j1), rows1, sg1)
        pltpu.async_copy(rows0, _sdst(j0), ss0, add=True)
        pltpu.make_async_copy(_gsrc(j1), rows1, sg1).wait()
        pltpu.make_async_copy(rows0, _sdst(j0), ss0).wait()
        pltpu.async_copy(_gsrc(j0 + 2), rows0, sg0)
        pltpu.async_copy(rows1, _sdst(j1), ss1, add=True)
        return carry
    lax.fori_loop(0, NPAIR, _pair, 0)

    jt = NCH - 1
    pltpu.make_async_copy(_gsrc(jt), rows0, sg0).wait()
    pltpu.make_async_copy(rows1, _sdst(jt - 1), ss1).wait()
    pltpu.async_copy(rows0, _sdst(jt), ss0, add=True)
    pltpu.make_async_copy(rows0, _sdst(jt), ss0).wait()
    plsc.subcore_barrier()

    @pl.when(s < N // WR)
    def _():
        sl = pl.ds(s * WR, WR)
        pltpu.sync_copy(acc.at[sl], out_hbm.at[c].at[sl])


def _sc_scatter(idx, dst, table_flat):
    mesh = plsc.VectorSubcoreMesh(core_axis_name="c", subcore_axis_name="s")
    f = functools.partial(
        pl.kernel,
        out_type=jax.ShapeDtypeStruct((NC, N, M), jnp.float32),
        mesh=mesh,
        scratch_types=[
            pltpu.VMEM((EPW,), jnp.int32),
            pltpu.VMEM((EPW,), jnp.int32),
            pltpu.VMEM((K, M), jnp.float32),
            pltpu.VMEM((K, M), jnp.float32),
            pltpu.VMEM((ZR, M), jnp.float32),
            pltpu.VMEM_SHARED((N, M), jnp.float32),
            pltpu.SemaphoreType.DMA,
            pltpu.SemaphoreType.DMA,
            pltpu.SemaphoreType.DMA,
            pltpu.SemaphoreType.DMA,
        ],
    )(_sc_body)
    return f(idx, dst, table_flat)


def _update_body(part_ref, x_ref, gid_ref, wih_ref, whh_ref, bih_ref,
                 bhh_ref, wi_ref, wj_ref, bj_ref, rd_ref, phis_ref):
    i = pl.program_id(0)
    m = part_ref[0] + part_ref[1]
    h = x_ref[...]

    gi = lax.dot_general(m, wih_ref[...], (((1,), (1,)), ((), ())),
                         preferred_element_type=jnp.float32) + bih_ref[0][None, :]
    gh = lax.dot_general(h, whh_ref[...], (((1,), (1,)), ((), ())),
                         preferred_element_type=jnp.float32) + bhh_ref[0][None, :]
    r = jax.nn.sigmoid(gi[:, :D] + gh[:, :D])
    z = jax.nn.sigmoid(gi[:, D:2 * D] + gh[:, D:2 * D])
    n = jnp.tanh(gi[:, 2 * D:] + r * gh[:, 2 * D:])
    hT = (1.0 - z) * n + z * h

    wi = wi_ref[...]
    wj = wj_ref[...]
    a = (lax.dot_general(hT, wi[:, :D], (((1,), (1,)), ((), ())),
                         preferred_element_type=jnp.float32)
         + lax.dot_general(h, wi[:, D:], (((1,), (1,)), ((), ())),
                           preferred_element_type=jnp.float32))
    b = (lax.dot_general(hT, wj[:, :D], (((1,), (1,)), ((), ())),
                         preferred_element_type=jnp.float32)
         + lax.dot_general(h, wj[:, D:], (((1,), (1,)), ((), ())),
                           preferred_element_type=jnp.float32)
         + bj_ref[0][None, :])
    rd = jax.nn.sigmoid(a) * jnp.tanh(b)
    rd_ref[...] = rd

    ids = gid_ref[...].reshape(1, BN)
    iota = lax.broadcasted_iota(jnp.int32, (NG, BN), 0)
    onehot = jnp.where(ids == iota, 1.0, 0.0)
    pb = lax.dot_general(onehot, rd, (((1,), (0,)), ((), ())),
                         preferred_element_type=jnp.float32)

    @pl.when(i == 0)
    def _init():
        phis_ref[...] = pb

    @pl.when(i > 0)
    def _acc():
        phis_ref[...] += pb


def _update(partials, x, gids3, W_ih, W_hh, b_ih, b_hh, W_i, W_j, b_j):
    full = lambda shape: pl.BlockSpec(shape, lambda i: tuple(0 for _ in shape))
    return pl.pallas_call(
        _update_body,
        grid=(NB,),
        in_specs=[
            pl.BlockSpec((NC, BN, M), lambda i: (0, i, 0)),
            pl.BlockSpec((BN, D), lambda i: (i, 0)),
            pl.BlockSpec((1, 1, BN), lambda i: (i, 0, 0)),
            full((3 * D, M)),
            full((3 * D, D)),
            full((1, 3 * D)),
            full((1, 3 * D)),
            full((D, 2 * D)),
            full((D, 2 * D)),
            full((1, D)),
        ],
        out_specs=[
            pl.BlockSpec((BN, D), lambda i: (i, 0)),
            pl.BlockSpec((NG, D), lambda i: (0, 0)),
        ],
        out_shape=[
            jax.ShapeDtypeStruct((N, D), jnp.float32),
            jax.ShapeDtypeStruct((NG, D), jnp.float32),
        ],
    )(partials, x, gids3, W_ih, W_hh, b_ih, b_hh, W_i, W_j, b_j)


def kernel(x, edge_index, edge_type, node_graph_ids, W_msg, b_msg,
           W_ih, W_hh, b_ih, b_hh, W_i, W_j, b_j):
    table = _build_msg_table(x, W_msg, b_msg)
    table_flat = table.reshape(T * N, M)

    idx = _build_idx(edge_index[0], edge_type)
    partials = _sc_scatter(idx, edge_index[1], table_flat)

    gids3 = node_graph_ids.reshape(NB, 1, BN)
    rd, phis = _update(partials, x, gids3, W_ih, W_hh,
                       b_ih.reshape(1, -1), b_hh.reshape(1, -1),
                       W_i, W_j, b_j.reshape(1, -1))
    return (rd, phis)


# speedup vs baseline: 35.0980x; 1.1379x over previous
import functools

import jax
import jax.numpy as jnp
from jax import lax
from jax.experimental import pallas as pl
from jax.experimental.pallas import tpu as pltpu
from jax.experimental.pallas import tpu_sc as plsc

N = 10000
E = 320000
D = 128
M = 128
T = 4
NG = 64

NC = 2
NS = 16
NW = NC * NS
EPW = E // NW
K = 128
NP1 = 20
NP2 = 19
S1E = 2 * NP1 * K
S2E = EPW - S1E
KT = S2E - 2 * NP2 * K
ZR = 80
NZC = N // ZR
WR = 2000
EB = E // 128

BN = 1000
NB = N // BN


def _msgs_body(x_ref, w_ref, b_ref, o_ref):
    t = pl.program_id(0)
    w = w_ref[0]
    b = b_ref[pl.ds(t, 1), :]
    acc = lax.dot_general(x_ref[...], w, (((1,), (1,)), ((), ())),
                          preferred_element_type=jnp.float32)
    o_ref[0] = acc + b


def _build_msg_table(x, W_msg, b_msg):
    return pl.pallas_call(
        _msgs_body,
        grid=(T, NB),
        in_specs=[
            pl.BlockSpec((BN, D), lambda t, i: (i, 0)),
            pl.BlockSpec((1, M, D), lambda t, i: (t, 0, 0)),
            pl.BlockSpec((T, M), lambda t, i: (0, 0)),
        ],
        out_specs=pl.BlockSpec((1, BN, M), lambda t, i: (t, i, 0)),
        out_shape=jax.ShapeDtypeStruct((T, N, M), jnp.float32),
    )(x, W_msg, b_msg)


def _idx_body(ei_ref, et_ref, o_ref):
    o_ref[...] = et_ref[...] * N + ei_ref[0]


def _build_idx(edge_index, et):
    out = pl.pallas_call(
        _idx_body,
        out_shape=jax.ShapeDtypeStruct((EB, 128), jnp.int32),
    )(edge_index.reshape(2, EB, 128), et.reshape(EB, 128))
    return out.reshape(E)


def _sc_body(idx_hbm, ei_hbm, table_hbm, out_hbm,
             idx_v, dst_v, rows0, rows1, acc,
             sg0, sg1, ss0, ss1):
    c = lax.axis_index("c")
    s = lax.axis_index("s")
    wid = s * NC + c
    base = wid * EPW

    pltpu.sync_copy(idx_hbm.at[pl.ds(base, EPW)], idx_v)
    pltpu.sync_copy(ei_hbm.at[pl.ds(E + base, S1E)], dst_v)

    def _zfill(i, carry):
        rows0[i // 8, pl.ds((i % 8) * 16, 16)] = jnp.zeros((16,), jnp.float32)
        return carry
    lax.fori_loop(0, K * 8, _zfill, 0)

    def _zacc(k, carry):
        cid = k * NS + s
        @pl.when(cid < NZC)
        def _():
            pltpu.sync_copy(rows0.at[pl.ds(0, ZR)], acc.at[pl.ds(cid * ZR, ZR)])
        return carry
    lax.fori_loop(0, (NZC + NS - 1) // NS, _zacc, 0)
    plsc.subcore_barrier()

    def _gsrc(j):
        return table_hbm.at[idx_v.at[pl.ds(j * K, K)]]

    def _segment(c0, npairs):
        def _gs(j):
            return _gsrc(c0 + j)

        def _sd(j):
            return acc.at[dst_v.at[pl.ds(j * K, K)]]

        pltpu.async_copy(_gs(0), rows0, sg0)

        def _pair(jj, carry):
            j0 = 2 * jj
            j1 = j0 + 1
            pltpu.make_async_copy(_gs(j0), rows0, sg0).wait()
            @pl.when(jj > 0)
            def _():
                pltpu.make_async_copy(rows1, _sd(j1 - 2), ss1).wait()
            pltpu.async_copy(_gs(j1), rows1, sg1)
            pltpu.async_copy(rows0, _sd(j0), ss0, add=True)
            pltpu.make_async_copy(_gs(j1), rows1, sg1).wait()
            pltpu.make_async_copy(rows0, _sd(j0), ss0).wait()
            @pl.when(j0 + 2 < 2 * npairs)
            def _():
                pltpu.async_copy(_gs(j0 + 2), rows0, sg0)
            pltpu.async_copy(rows1, _sd(j1), ss1, add=True)
            return carry
        lax.fori_loop(0, npairs, _pair, 0)
        pltpu.make_async_copy(rows1, _sd(2 * npairs - 1), ss1).wait()

    _segment(0, NP1)

    pltpu.sync_copy(ei_hbm.at[pl.ds(E + base + S1E, S2E)], dst_v.at[pl.ds(0, S2E)])
    _segment(2 * NP1, NP2)

    tidx = idx_v.at[pl.ds(EPW - KT, KT)]
    tdst = dst_v.at[pl.ds(S2E - KT, KT)]
    rt = rows0.at[pl.ds(0, KT)]
    pltpu.async_copy(table_hbm.at[tidx], rt, sg0)
    pltpu.make_async_copy(table_hbm.at[tidx], rt, sg0).wait()
    pltpu.async_copy(rt, acc.at[tdst], ss0, add=True)
    pltpu.make_async_copy(rt, acc.at[tdst], ss0).wait()
    plsc.subcore_barrier()

    @pl.when(s < N // WR)
    def _():
        sl = pl.ds(s * WR, WR)
        pltpu.sync_copy(acc.at[sl], out_hbm.at[c].at[sl])


def _sc_scatter(idx, ei_flat, table_flat):
    mesh = plsc.VectorSubcoreMesh(core_axis_name="c", subcore_axis_name="s")
    f = functools.partial(
        pl.kernel,
        out_type=jax.ShapeDtypeStruct((NC, N, M), jnp.float32),
        mesh=mesh,
        scratch_types=[
            pltpu.VMEM((EPW,), jnp.int32),
            pltpu.VMEM((S1E,), jnp.int32),
            pltpu.VMEM((K, M), jnp.float32),
            pltpu.VMEM((K, M), jnp.float32),
            pltpu.VMEM_SHARED((N, M), jnp.float32),
            pltpu.SemaphoreType.DMA,
            pltpu.SemaphoreType.DMA,
            pltpu.SemaphoreType.DMA,
            pltpu.SemaphoreType.DMA,
        ],
    )(_sc_body)
    return f(idx, ei_flat, table_flat)


def _update_body(part_ref, x_ref, gid_ref, wih_ref, whh_ref, bih_ref,
                 bhh_ref, wi_ref, wj_ref, bj_ref, rd_ref, phis_ref):
    i = pl.program_id(0)
    m = part_ref[0] + part_ref[1]
    h = x_ref[...]

    gi = lax.dot_general(m, wih_ref[...], (((1,), (1,)), ((), ())),
                         preferred_element_type=jnp.float32) + bih_ref[0][None, :]
    gh = lax.dot_general(h, whh_ref[...], (((1,), (1,)), ((), ())),
                         preferred_element_type=jnp.float32) + bhh_ref[0][None, :]
    r = jax.nn.sigmoid(gi[:, :D] + gh[:, :D])
    z = jax.nn.sigmoid(gi[:, D:2 * D] + gh[:, D:2 * D])
    n = jnp.tanh(gi[:, 2 * D:] + r * gh[:, 2 * D:])
    hT = (1.0 - z) * n + z * h

    wi = wi_ref[...]
    wj = wj_ref[...]
    a = (lax.dot_general(hT, wi[:, :D], (((1,), (1,)), ((), ())),
                         preferred_element_type=jnp.float32)
         + lax.dot_general(h, wi[:, D:], (((1,), (1,)), ((), ())),
                           preferred_element_type=jnp.float32))
    b = (lax.dot_general(hT, wj[:, :D], (((1,), (1,)), ((), ())),
                         preferred_element_type=jnp.float32)
         + lax.dot_general(h, wj[:, D:], (((1,), (1,)), ((), ())),
                           preferred_element_type=jnp.float32)
         + bj_ref[0][None, :])
    rd = jax.nn.sigmoid(a) * jnp.tanh(b)
    rd_ref[...] = rd

    ids = gid_ref[...].reshape(1, BN)
    iota = lax.broadcasted_iota(jnp.int32, (NG, BN), 0)
    onehot = jnp.where(ids == iota, 1.0, 0.0)
    pb = lax.dot_general(onehot, rd, (((1,), (0,)), ((), ())),
                         preferred_element_type=jnp.float32)

    @pl.when(i == 0)
    def _init():
        phis_ref[...] = pb

    @pl.when(i > 0)
    def _acc():
        phis_ref[...] += pb


def _update(partials, x, gids3, W_ih, W_hh, b_ih, b_hh, W_i, W_j, b_j):
    full = lambda shape: pl.BlockSpec(shape, lambda i: tuple(0 for _ in shape))
    return pl.pallas_call(
        _update_body,
        grid=(NB,),
        in_specs=[
            pl.BlockSpec((NC, BN, M), lambda i: (0, i, 0)),
            pl.BlockSpec((BN, D), lambda i: (i, 0)),
            pl.BlockSpec((1, 1, BN), lambda i: (i, 0, 0)),
            full((3 * D, M)),
            full((3 * D, D)),
            full((1, 3 * D)),
            full((1, 3 * D)),
            full((D, 2 * D)),
            full((D, 2 * D)),
            full((1, D)),
        ],
        out_specs=[
            pl.BlockSpec((BN, D), lambda i: (i, 0)),
            pl.BlockSpec((NG, D), lambda i: (0, 0)),
        ],
        out_shape=[
            jax.ShapeDtypeStruct((N, D), jnp.float32),
            jax.ShapeDtypeStruct((NG, D), jnp.float32),
        ],
    )(partials, x, gids3, W_ih, W_hh, b_ih, b_hh, W_i, W_j, b_j)


def kernel(x, edge_index, edge_type, node_graph_ids, W_msg, b_msg,
           W_ih, W_hh, b_ih, b_hh, W_i, W_j, b_j):
    table = _build_msg_table(x, W_msg, b_msg)
    table_flat = table.reshape(T * N, M)

    idx = _build_idx(edge_index, edge_type)
    partials = _sc_scatter(idx, edge_index.reshape(2 * E), table_flat)

    gids3 = node_graph_ids.reshape(NB, 1, BN)
    rd, phis = _update(partials, x, gids3, W_ih, W_hh,
                       b_ih.reshape(1, -1), b_hh.reshape(1, -1),
                       W_i, W_j, b_j.reshape(1, -1))
    return (rd, phis)


# speedup vs baseline: 35.9127x; 1.0232x over previous
import functools

import jax
import jax.numpy as jnp
from jax import lax
from jax.experimental import pallas as pl
from jax.experimental.pallas import tpu as pltpu
from jax.experimental.pallas import tpu_sc as plsc

N = 10000
E = 320000
D = 128
M = 128
T = 4
NG = 64

NC = 2
NS = 16
NW = NC * NS
EPW = E // NW
K = 128
NP1 = 20
NP2 = 19
S1E = 2 * NP1 * K
S2E = EPW - S1E
KT = S2E - 2 * NP2 * K
ZR = 80
NZC = N // ZR
WR = 2000
EB = E // 128

BN = 1000
NB = N // BN


EBB = EB // NB


def _msgs_body(x_ref, w_ref, b_ref, ei_ref, et_ref, o_ref, oi_ref):
    i = pl.program_id(0)
    t = pl.program_id(1)
    w = w_ref[0]
    b = b_ref[pl.ds(t, 1), :]
    acc = lax.dot_general(x_ref[...], w, (((1,), (1,)), ((), ())),
                          preferred_element_type=jnp.float32)
    o_ref[0] = acc + b

    @pl.when((t == 0) & (i == 0))
    def _():
        oi_ref[...] = et_ref[...] * N + ei_ref[0]


def _build_msg_table(x, W_msg, b_msg, edge_index, et):
    return pl.pallas_call(
        _msgs_body,
        grid=(NB, T),
        in_specs=[
            pl.BlockSpec((BN, D), lambda i, t: (i, 0)),
            pl.BlockSpec((1, M, D), lambda i, t: (t, 0, 0)),
            pl.BlockSpec((T, M), lambda i, t: (0, 0)),
            pl.BlockSpec((2, EB, 128), lambda i, t: (0, 0, 0)),
            pl.BlockSpec((EB, 128), lambda i, t: (0, 0)),
        ],
        out_specs=[
            pl.BlockSpec((1, BN, M), lambda i, t: (t, i, 0)),
            pl.BlockSpec((EB, 128), lambda i, t: (0, 0)),
        ],
        out_shape=[
            jax.ShapeDtypeStruct((T, N, M), jnp.float32),
            jax.ShapeDtypeStruct((EB, 128), jnp.int32),
        ],
    )(x, W_msg, b_msg, edge_index.reshape(2, EB, 128), et.reshape(EB, 128))


def _sc_body(idx_hbm, ei_hbm, table_hbm, out_hbm,
             idx_v, dst_v, rows0, rows1, acc,
             sg0, sg1, ss0, ss1):
    c = lax.axis_index("c")
    s = lax.axis_index("s")
    wid = s * NC + c
    base = wid * EPW

    pltpu.sync_copy(idx_hbm.at[pl.ds(base, EPW)], idx_v)
    pltpu.sync_copy(ei_hbm.at[pl.ds(E + base, S1E)], dst_v)

    def _zfill(i, carry):
        rows0[i // 8, pl.ds((i % 8) * 16, 16)] = jnp.zeros((16,), jnp.float32)
        return carry
    lax.fori_loop(0, K * 8, _zfill, 0)

    def _zacc(k, carry):
        cid = k * NS + s
        @pl.when(cid < NZC)
        def _():
            pltpu.sync_copy(rows0.at[pl.ds(0, ZR)], acc.at[pl.ds(cid * ZR, ZR)])
        return carry
    lax.fori_loop(0, (NZC + NS - 1) // NS, _zacc, 0)
    plsc.subcore_barrier()

    def _gsrc(j):
        return table_hbm.at[idx_v.at[pl.ds(j * K, K)]]

    def _segment(c0, npairs):
        def _gs(j):
            return _gsrc(c0 + j)

        def _sd(j):
            return acc.at[dst_v.at[pl.ds(j * K, K)]]

        pltpu.async_copy(_gs(0), rows0, sg0)

        def _pair(jj, carry):
            j0 = 2 * jj
            j1 = j0 + 1
            pltpu.make_async_copy(_gs(j0), rows0, sg0).wait()
            @pl.when(jj > 0)
            def _():
                pltpu.make_async_copy(rows1, _sd(j1 - 2), ss1).wait()
            pltpu.async_copy(_gs(j1), rows1, sg1)
            pltpu.async_copy(rows0, _sd(j0), ss0, add=True)
            pltpu.make_async_copy(_gs(j1), rows1, sg1).wait()
            pltpu.make_async_copy(rows0, _sd(j0), ss0).wait()
            @pl.when(j0 + 2 < 2 * npairs)
            def _():
                pltpu.async_copy(_gs(j0 + 2), rows0, sg0)
            pltpu.async_copy(rows1, _sd(j1), ss1, add=True)
            return carry
        lax.fori_loop(0, npairs, _pair, 0)
        pltpu.make_async_copy(rows1, _sd(2 * npairs - 1), ss1).wait()

    _segment(0, NP1)

    pltpu.sync_copy(ei_hbm.at[pl.ds(E + base + S1E, S2E)], dst_v.at[pl.ds(0, S2E)])
    _segment(2 * NP1, NP2)

    tidx = idx_v.at[pl.ds(EPW - KT, KT)]
    tdst = dst_v.at[pl.ds(S2E - KT, KT)]
    rt = rows0.at[pl.ds(0, KT)]
    pltpu.async_copy(table_hbm.at[tidx], rt, sg0)
    pltpu.make_async_copy(table_hbm.at[tidx], rt, sg0).wait()
    pltpu.async_copy(rt, acc.at[tdst], ss0, add=True)
    pltpu.make_async_copy(rt, acc.at[tdst], ss0).wait()
    plsc.subcore_barrier()

    @pl.when(s < N // WR)
    def _():
        sl = pl.ds(s * WR, WR)
        pltpu.sync_copy(acc.at[sl], out_hbm.at[c].at[sl])


def _sc_scatter(idx, ei_flat, table_flat):
    mesh = plsc.VectorSubcoreMesh(core_axis_name="c", subcore_axis_name="s")
    f = functools.partial(
        pl.kernel,
        out_type=jax.ShapeDtypeStruct((NC, N, M), jnp.float32),
        mesh=mesh,
        scratch_types=[
            pltpu.VMEM((EPW,), jnp.int32),
            pltpu.VMEM((S1E,), jnp.int32),
            pltpu.VMEM((K, M), jnp.float32),
            pltpu.VMEM((K, M), jnp.float32),
            pltpu.VMEM_SHARED((N, M), jnp.float32),
            pltpu.SemaphoreType.DMA,
            pltpu.SemaphoreType.DMA,
            pltpu.SemaphoreType.DMA,
            pltpu.SemaphoreType.DMA,
        ],
    )(_sc_body)
    return f(idx, ei_flat, table_flat)


def _update_body(part_ref, x_ref, gid_ref, wih_ref, whh_ref, bih_ref,
                 bhh_ref, wi_ref, wj_ref, bj_ref, rd_ref, phis_ref):
    i = pl.program_id(0)
    m = part_ref[0] + part_ref[1]
    h = x_ref[...]

    gi = lax.dot_general(m, wih_ref[...], (((1,), (1,)), ((), ())),
                         preferred_element_type=jnp.float32) + bih_ref[0][None, :]
    gh = lax.dot_general(h, whh_ref[...], (((1,), (1,)), ((), ())),
                         preferred_element_type=jnp.float32) + bhh_ref[0][None, :]
    r = jax.nn.sigmoid(gi[:, :D] + gh[:, :D])
    z = jax.nn.sigmoid(gi[:, D:2 * D] + gh[:, D:2 * D])
    n = jnp.tanh(gi[:, 2 * D:] + r * gh[:, 2 * D:])
    hT = (1.0 - z) * n + z * h

    wi = wi_ref[...]
    wj = wj_ref[...]
    a = (lax.dot_general(hT, wi[:, :D], (((1,), (1,)), ((), ())),
                         preferred_element_type=jnp.float32)
         + lax.dot_general(h, wi[:, D:], (((1,), (1,)), ((), ())),
                           preferred_element_type=jnp.float32))
    b = (lax.dot_general(hT, wj[:, :D], (((1,), (1,)), ((), ())),
                         preferred_element_type=jnp.float32)
         + lax.dot_general(h, wj[:, D:], (((1,), (1,)), ((), ())),
                           preferred_element_type=jnp.float32)
         + bj_ref[0][None, :])
    rd = jax.nn.sigmoid(a) * jnp.tanh(b)
    rd_ref[...] = rd

    ids = gid_ref[...].reshape(1, BN)
    iota = lax.broadcasted_iota(jnp.int32, (NG, BN), 0)
    onehot = jnp.where(ids == iota, 1.0, 0.0)
    pb = lax.dot_general(onehot, rd, (((1,), (0,)), ((), ())),
                         preferred_element_type=jnp.float32)

    @pl.when(i == 0)
    def _init():
        phis_ref[...] = pb

    @pl.when(i > 0)
    def _acc():
        phis_ref[...] += pb


def _update(partials, x, gids3, W_ih, W_hh, b_ih, b_hh, W_i, W_j, b_j):
    full = lambda shape: pl.BlockSpec(shape, lambda i: tuple(0 for _ in shape))
    return pl.pallas_call(
        _update_body,
        grid=(NB,),
        in_specs=[
            pl.BlockSpec((NC, BN, M), lambda i: (0, i, 0)),
            pl.BlockSpec((BN, D), lambda i: (i, 0)),
            pl.BlockSpec((1, 1, BN), lambda i: (i, 0, 0)),
            full((3 * D, M)),
            full((3 * D, D)),
            full((1, 3 * D)),
            full((1, 3 * D)),
            full((D, 2 * D)),
            full((D, 2 * D)),
            full((1, D)),
        ],
        out_specs=[
            pl.BlockSpec((BN, D), lambda i: (i, 0)),
            pl.BlockSpec((NG, D), lambda i: (0, 0)),
        ],
        out_shape=[
            jax.ShapeDtypeStruct((N, D), jnp.float32),
            jax.ShapeDtypeStruct((NG, D), jnp.float32),
        ],
    )(partials, x, gids3, W_ih, W_hh, b_ih, b_hh, W_i, W_j, b_j)


def kernel(x, edge_index, edge_type, node_graph_ids, W_msg, b_msg,
           W_ih, W_hh, b_ih, b_hh, W_i, W_j, b_j):
    table, idx = _build_msg_table(x, W_msg, b_msg, edge_index, edge_type)
    table_flat = table.reshape(T * N, M)
    partials = _sc_scatter(idx.reshape(E), edge_index.reshape(2 * E),
                           table_flat)

    gids3 = node_graph_ids.reshape(NB, 1, BN)
    rd, phis = _update(partials, x, gids3, W_ih, W_hh,
                       b_ih.reshape(1, -1), b_hh.reshape(1, -1),
                       W_i, W_j, b_j.reshape(1, -1))
    return (rd, phis)
